# baseline (device time: 25746 ns/iter reference)
import os

import jax
import jax.numpy as jnp
from jax import lax
from jax.experimental import pallas as pl
from jax.experimental.pallas import tpu as pltpu

N_DEV = 8

ORDER = os.environ.get("A2A_ORDER", "xor")
CHUNKS = int(os.environ.get("A2A_CHUNKS", "1"))

_DELTAS = [
    (1, 0, 0), (0, 1, 0), (0, 0, 1),
    (1, 1, 0), (0, 1, 1), (1, 0, 1),
    (1, 1, 1),
]


def _xor_partner(me, delta):
    dx, dy, dz = delta
    z = me // 4
    r = me - 4 * z
    x = jnp.where((r == 1) | (r == 2), 1, 0)
    y = jnp.where(r >= 2, 1, 0)
    px = x + dx - 2 * x * dx
    py = y + dy - 2 * y * dy
    pz = z + dz - 2 * z * dz
    p4 = 3 * py + px - 2 * px * py
    return 4 * pz + p4


def kernel(x):
    m, n_total = x.shape
    blk = n_total // N_DEV
    out_rows = m * N_DEV
    mc = m // CHUNKS

    def body(x_ref, out_ref, send_sems, recv_sems):
        me = lax.axis_index("i")

        rdmas = []
        for k in range(1, N_DEV):
            if ORDER == "xor":
                j = _xor_partner(me, _DELTAS[k - 1])
            else:
                j = lax.rem(me + k, N_DEV)
            for c in range(CHUNKS):
                s = (k - 1) * CHUNKS + c
                rdma = pltpu.make_async_remote_copy(
                    src_ref=x_ref.at[pl.ds(c * mc, mc), pl.ds(j * blk, blk)],
                    dst_ref=out_ref.at[pl.ds(me * m + c * mc, mc), :],
                    send_sem=send_sems.at[s],
                    recv_sem=recv_sems.at[s],
                    device_id=(j,),
                    device_id_type=pl.DeviceIdType.MESH,
                )
                rdma.start()
                rdmas.append(rdma)

        out_ref[pl.ds(me * m, m), :] = x_ref[:, pl.ds(me * blk, blk)]

        for rdma in rdmas:
            rdma.wait()

    nsem = (N_DEV - 1) * CHUNKS
    return pl.pallas_call(
        body,
        out_shape=jax.ShapeDtypeStruct((out_rows, blk), x.dtype),
        in_specs=[pl.BlockSpec(memory_space=pltpu.VMEM)],
        out_specs=pl.BlockSpec(memory_space=pltpu.VMEM),
        scratch_shapes=[
            pltpu.SemaphoreType.DMA((nsem,)),
            pltpu.SemaphoreType.DMA((nsem,)),
        ],
    )(x)


# device time: 14532 ns/iter; 1.7717x vs baseline; 1.7717x over previous
import os

import jax
import jax.numpy as jnp
from jax import lax
from jax.experimental import pallas as pl
from jax.experimental.pallas import tpu as pltpu

N_DEV = 8

ORDER = os.environ.get("A2A_ORDER", "xor")
CHUNKS = int(os.environ.get("A2A_CHUNKS", "1"))
NSENDS = int(os.environ.get("A2A_NSENDS", "7"))

_DELTAS = [
    (1, 0, 0), (0, 1, 0), (0, 0, 1),
    (1, 1, 0), (0, 1, 1), (1, 0, 1),
    (1, 1, 1),
]


def _xor_partner(me, delta):
    dx, dy, dz = delta
    z = me // 4
    r = me - 4 * z
    x = jnp.where((r == 1) | (r == 2), 1, 0)
    y = jnp.where(r >= 2, 1, 0)
    px = x + dx - 2 * x * dx
    py = y + dy - 2 * y * dy
    pz = z + dz - 2 * z * dz
    p4 = 3 * py + px - 2 * px * py
    return 4 * pz + p4


def kernel(x):
    m, n_total = x.shape
    blk = n_total // N_DEV
    out_rows = m * N_DEV
    mc = m // CHUNKS

    def body(x_ref, out_ref, send_sems, recv_sems):
        me = lax.axis_index("i")

        rdmas = []
        for k in range(1, 1 + NSENDS):
            if ORDER == "xor":
                j = _xor_partner(me, _DELTAS[k - 1])
            else:
                j = lax.rem(me + k, N_DEV)
            for c in range(CHUNKS):
                s = (k - 1) * CHUNKS + c
                rdma = pltpu.make_async_remote_copy(
                    src_ref=x_ref.at[pl.ds(c * mc, mc), pl.ds(j * blk, blk)],
                    dst_ref=out_ref.at[pl.ds(me * m + c * mc, mc), :],
                    send_sem=send_sems.at[s],
                    recv_sem=recv_sems.at[s],
                    device_id=(j,),
                    device_id_type=pl.DeviceIdType.MESH,
                )
                rdma.start()
                rdmas.append(rdma)

        out_ref[pl.ds(me * m, m), :] = x_ref[:, pl.ds(me * blk, blk)]

        for rdma in rdmas:
            rdma.wait()

    nsem = (N_DEV - 1) * CHUNKS
    return pl.pallas_call(
        body,
        out_shape=jax.ShapeDtypeStruct((out_rows, blk), x.dtype),
        in_specs=[pl.BlockSpec(memory_space=pltpu.VMEM)],
        out_specs=pl.BlockSpec(memory_space=pltpu.VMEM),
        scratch_shapes=[
            pltpu.SemaphoreType.DMA((nsem,)),
            pltpu.SemaphoreType.DMA((nsem,)),
        ],
    )(x)


# device time: 13020 ns/iter; 1.9774x vs baseline; 1.1161x over previous
import os

import jax
import jax.numpy as jnp
from jax import lax
from jax.experimental import pallas as pl
from jax.experimental.pallas import tpu as pltpu

N_DEV = 8

ORDER = os.environ.get("A2A_ORDER", "xor")
CHUNKS = int(os.environ.get("A2A_CHUNKS", "1"))
NSENDS = int(os.environ.get("A2A_NSENDS", "7"))
HALF = int(os.environ.get("A2A_HALF", "1"))

_DELTAS = [
    (1, 0, 0), (0, 1, 0), (0, 0, 1),
    (1, 1, 0), (0, 1, 1), (1, 0, 1),
    (1, 1, 1),
]


def _xor_partner(me, delta):
    dx, dy, dz = delta
    z = me // 4
    r = me - 4 * z
    x = jnp.where((r == 1) | (r == 2), 1, 0)
    y = jnp.where(r >= 2, 1, 0)
    px = x + dx - 2 * x * dx
    py = y + dy - 2 * y * dy
    pz = z + dz - 2 * z * dz
    p4 = 3 * py + px - 2 * px * py
    return 4 * pz + p4


def kernel(x):
    m, n_total = x.shape
    blk = n_total // N_DEV
    out_rows = m * N_DEV
    mc = m // CHUNKS // HALF

    def body(x_ref, out_ref, send_sems, recv_sems):
        me = lax.axis_index("i")

        rdmas = []
        for k in range(1, 1 + NSENDS):
            if ORDER == "xor":
                j = _xor_partner(me, _DELTAS[k - 1])
            else:
                j = lax.rem(me + k, N_DEV)
            for c in range(CHUNKS):
                s = (k - 1) * CHUNKS + c
                rdma = pltpu.make_async_remote_copy(
                    src_ref=x_ref.at[pl.ds(c * mc, mc), pl.ds(j * blk, blk)],
                    dst_ref=out_ref.at[pl.ds(me * m + c * mc, mc), :],
                    send_sem=send_sems.at[s],
                    recv_sem=recv_sems.at[s],
                    device_id=(j,),
                    device_id_type=pl.DeviceIdType.MESH,
                )
                rdma.start()
                rdmas.append(rdma)

        out_ref[pl.ds(me * m, m), :] = x_ref[:, pl.ds(me * blk, blk)]

        for rdma in rdmas:
            rdma.wait()

    nsem = (N_DEV - 1) * CHUNKS
    return pl.pallas_call(
        body,
        out_shape=jax.ShapeDtypeStruct((out_rows, blk), x.dtype),
        in_specs=[pl.BlockSpec(memory_space=pltpu.VMEM)],
        out_specs=pl.BlockSpec(memory_space=pltpu.VMEM),
        scratch_shapes=[
            pltpu.SemaphoreType.DMA((nsem,)),
            pltpu.SemaphoreType.DMA((nsem,)),
        ],
    )(x)


# device time: 7883 ns/iter; 3.2660x vs baseline; 1.6517x over previous
import os

import jax
import jax.numpy as jnp
from jax import lax
from jax.experimental import pallas as pl
from jax.experimental.pallas import tpu as pltpu

N_DEV = 8

ORDER = os.environ.get("A2A_ORDER", "xor")
CHUNKS = int(os.environ.get("A2A_CHUNKS", "1"))
NSENDS = int(os.environ.get("A2A_NSENDS", "7"))
HALF = int(os.environ.get("A2A_HALF", "1"))
BARRIER = os.environ.get("A2A_BARRIER", "auto")

_DELTAS = [
    (1, 0, 0), (0, 1, 0), (0, 0, 1),
    (1, 1, 0), (0, 1, 1), (1, 0, 1),
    (1, 1, 1),
]


def _xor_partner(me, delta):
    dx, dy, dz = delta
    z = me // 4
    r = me - 4 * z
    x = jnp.where((r == 1) | (r == 2), 1, 0)
    y = jnp.where(r >= 2, 1, 0)
    px = x + dx - 2 * x * dx
    py = y + dy - 2 * y * dy
    pz = z + dz - 2 * z * dz
    p4 = 3 * py + px - 2 * px * py
    return 4 * pz + p4


def kernel(x):
    m, n_total = x.shape
    blk = n_total // N_DEV
    out_rows = m * N_DEV
    mc = m // CHUNKS // HALF

    def body(x_ref, out_ref, send_sems, recv_sems):
        me = lax.axis_index("i")

        if BARRIER == "explicit":
            barrier_sem = pltpu.get_barrier_semaphore()
            for k in range(1, N_DEV):
                nbr = lax.rem(me + k, N_DEV)
                pl.semaphore_signal(
                    barrier_sem, inc=1,
                    device_id=(nbr,), device_id_type=pl.DeviceIdType.MESH,
                )
            pl.semaphore_wait(barrier_sem, N_DEV - 1)

        rdmas = []
        for k in range(1, 1 + NSENDS):
            if ORDER == "xor":
                j = _xor_partner(me, _DELTAS[k - 1])
            else:
                j = lax.rem(me + k, N_DEV)
            for c in range(CHUNKS):
                s = (k - 1) * CHUNKS + c
                rdma = pltpu.make_async_remote_copy(
                    src_ref=x_ref.at[pl.ds(c * mc, mc), pl.ds(j * blk, blk)],
                    dst_ref=out_ref.at[pl.ds(me * m + c * mc, mc), :],
                    send_sem=send_sems.at[s],
                    recv_sem=recv_sems.at[s],
                    device_id=(j,),
                    device_id_type=pl.DeviceIdType.MESH,
                )
                rdma.start()
                rdmas.append(rdma)

        out_ref[pl.ds(me * m, m), :] = x_ref[:, pl.ds(me * blk, blk)]

        for rdma in rdmas:
            rdma.wait()

    nsem = (N_DEV - 1) * CHUNKS
    return pl.pallas_call(
        body,
        out_shape=jax.ShapeDtypeStruct((out_rows, blk), x.dtype),
        in_specs=[pl.BlockSpec(memory_space=pltpu.VMEM)],
        out_specs=pl.BlockSpec(memory_space=pltpu.VMEM),
        scratch_shapes=[
            pltpu.SemaphoreType.DMA((nsem,)),
            pltpu.SemaphoreType.DMA((nsem,)),
        ],
        compiler_params=pltpu.CompilerParams(collective_id=0),
    )(x)
